# Initial kernel scaffold; baseline (speedup 1.0000x reference)
import jax
import jax.numpy as jnp
from jax.experimental import pallas as pl


def kernel(item, edge_index, user_embedding_weight, source_item_embedding_weight, W0, b0, W1, b1, Wl, bl, domain_prior):
    raise NotImplementedError("write your pallas kernel here")


# speedup vs baseline: 71.6066x; 71.6066x over previous
import jax
import jax.numpy as jnp
from jax import lax
from jax.experimental import pallas as pl
from jax.experimental.pallas import tpu as pltpu
from jax.experimental.pallas import tpu_sc as plsc

_f32 = jnp.float32
_i32 = jnp.int32

D = 128
N_USERS = 4000
N_ITEMS = 6000
N = N_USERS + N_ITEMS
E = 320000
NPAD = 10240
NSUB = 16
NT = NPAD // NSUB
ECH = 1256
EW = ECH * 16
EPAD = EW * NSUB
ICH = N_ITEMS // 16


def _dense_body(x_ref, w0_ref, w1_ref, wl_ref, b0_ref, b1_ref, bl_ref,
                z_ref, betas_ref):
    w2 = jnp.dot(w1_ref[...], wl_ref[...], preferred_element_type=_f32)
    w1v = jnp.dot(w0_ref[...], w2, preferred_element_type=_f32)
    z_ref[...] = jnp.dot(x_ref[...], w1v, preferred_element_type=_f32)
    beta1 = jnp.dot(b0_ref[...], w2, preferred_element_type=_f32)
    beta2 = jnp.dot(b1_ref[...], wl_ref[...], preferred_element_type=_f32)
    beta2 = beta2 + bl_ref[...]
    betas_ref[...] = jnp.concatenate(
        [jnp.broadcast_to(beta1, (1, D)), jnp.broadcast_to(beta2, (1, D))],
        axis=0)


def _fast_rsqrt(d):
    i = plsc.bitcast(d, _i32)
    y = plsc.bitcast(jnp.int32(0x5F3759DF) - (i >> 1), _f32)
    for _ in range(3):
        y = y * (1.5 - 0.5 * d * y * y)
    return y


def _sc_body(z_hbm, src_hbm, dst_hbm, item_hbm, betas_hbm,
             out_hbm,
             esrc, edst, tbl, acc, ptmp, rsl, zsl, dinvsl, thsl, vhsl, bb,
             items_v, res_v, sh_all, sh_tbl):
    t = lax.axis_index("s")
    core = lax.axis_index("c")
    ones = jnp.full((16,), 1.0, _f32)

    pltpu.sync_copy(src_hbm.at[pl.ds(t * EW, EW)], esrc)
    pltpu.sync_copy(dst_hbm.at[pl.ds(t * EW, EW)], edst)
    pltpu.sync_copy(z_hbm.at[pl.ds(t * NT, NT)], zsl)
    pltpu.sync_copy(betas_hbm, bb)

    def zero_acc():
        def zb(r, c):
            acc[pl.ds(r * 16, 16)] = jnp.zeros((16,), _f32)
            return c
        lax.fori_loop(0, NPAD // 16, zb, 0)

    def scatter_pass(gather_from_tbl):
        zero_acc()

        def eb(i, c):
            di = edst[pl.ds(i * 16, 16)]
            if gather_from_tbl:
                si = esrc[pl.ds(i * 16, 16)]
                vals = plsc.load_gather(tbl, [si])
            else:
                vals = ones
            plsc.addupdate_scatter(acc, [di], vals)
            return c
        lax.fori_loop(0, ECH, eb, 0)

    def reduce_acc():
        pltpu.sync_copy(acc, sh_all.at[t])
        plsc.subcore_barrier()
        pltpu.sync_copy(sh_all.at[0, pl.ds(t * NT, NT)], rsl)

        def rb(p, c):
            pltpu.sync_copy(sh_all.at[p, pl.ds(t * NT, NT)], ptmp)

            def ab(r, c2):
                o = r * 16
                rsl[pl.ds(o, 16)] = rsl[pl.ds(o, 16)] + ptmp[pl.ds(o, 16)]
                return c2
            lax.fori_loop(0, NT // 16, ab, 0)
            return c
        lax.fori_loop(1, NSUB, rb, 0)
        plsc.subcore_barrier()

    def publish_and_reload(slice_ref):
        pltpu.sync_copy(slice_ref, sh_tbl.at[pl.ds(t * NT, NT)])
        plsc.subcore_barrier()
        pltpu.sync_copy(sh_tbl, tbl)

    scatter_pass(gather_from_tbl=False)
    reduce_acc()

    def dnb(r, c):
        o = r * 16
        dv = _fast_rsqrt(rsl[pl.ds(o, 16)] + 1.0)
        dinvsl[pl.ds(o, 16)] = dv
        thsl[pl.ds(o, 16)] = zsl[pl.ds(o, 16)] * dv
        return c
    lax.fori_loop(0, NT // 16, dnb, 0)
    publish_and_reload(thsl)

    scatter_pass(gather_from_tbl=True)
    reduce_acc()
    b1v = bb[pl.ds(0, 16)]

    def vnb(r, c):
        o = r * 16
        vv = dinvsl[pl.ds(o, 16)] * (rsl[pl.ds(o, 16)] + thsl[pl.ds(o, 16)])
        vv = vv + b1v
        vhsl[pl.ds(o, 16)] = vv * dinvsl[pl.ds(o, 16)]
        return c
    lax.fori_loop(0, NT // 16, vnb, 0)
    publish_and_reload(vhsl)

    scatter_pass(gather_from_tbl=True)
    reduce_acc()
    b2v = bb[pl.ds(16, 16)]

    def snb(r, c):
        o = r * 16
        sv = dinvsl[pl.ds(o, 16)] * (rsl[pl.ds(o, 16)] + vhsl[pl.ds(o, 16)])
        thsl[pl.ds(o, 16)] = sv + b2v
        return c
    lax.fori_loop(0, NT // 16, snb, 0)
    pltpu.sync_copy(thsl, sh_tbl.at[pl.ds(t * NT, NT)])
    plsc.subcore_barrier()

    @pl.when(jnp.logical_and(t == 0, core == 0))
    def _final():
        pltpu.sync_copy(sh_tbl, tbl)
        pltpu.sync_copy(item_hbm, items_v)

        def g1(i, m):
            it = items_v[pl.ds(i * 16, 16)]
            sv = plsc.load_gather(tbl, [it])
            res_v[pl.ds(i * 16, 16)] = sv
            return jnp.maximum(m, sv)
        m = lax.fori_loop(0, ICH, g1, jnp.full((16,), -1e30, _f32))
        gm = jnp.max(m)

        def g2(i, ssum):
            o = i * 16
            e = jnp.exp(res_v[pl.ds(o, 16)] - gm)
            res_v[pl.ds(o, 16)] = e
            return ssum + e
        ssum = lax.fori_loop(0, ICH, g2, jnp.zeros((16,), _f32))
        invv = jnp.ones((16,), _f32) / jnp.broadcast_to(jnp.sum(ssum), (16,))

        def g3(i, c):
            o = i * 16
            res_v[pl.ds(o, 16)] = res_v[pl.ds(o, 16)] * invv
            return c
        lax.fori_loop(0, ICH, g3, 0)
        pltpu.sync_copy(res_v, out_hbm)


def kernel(item, edge_index, user_embedding_weight, source_item_embedding_weight,
           W0, b0, W1, b1, Wl, bl, domain_prior):
    x = jnp.concatenate([user_embedding_weight, source_item_embedding_weight],
                        axis=0)
    z, betas = pl.pallas_call(
        _dense_body,
        out_shape=[jax.ShapeDtypeStruct((N, 1), _f32),
                   jax.ShapeDtypeStruct((2, D), _f32)],
    )(x, W0, W1, Wl, b0.reshape(1, D), b1.reshape(1, D), bl.reshape(1, 1))

    z1d = jnp.pad(z[:, 0], (0, NPAD - N))
    src1d = jnp.pad(edge_index[0].astype(_i32), (0, EPAD - E),
                    constant_values=N)
    dst1d = jnp.pad(edge_index[1].astype(_i32), (0, EPAD - E),
                    constant_values=N)
    item1d = item.astype(_i32)
    betas1d = betas[:, :16].reshape(32)

    sc = pl.kernel(
        _sc_body,
        out_type=jax.ShapeDtypeStruct((N_ITEMS,), _f32),
        mesh=plsc.VectorSubcoreMesh(core_axis_name="c", subcore_axis_name="s"),
        compiler_params=pltpu.CompilerParams(needs_layout_passes=False),
        scratch_types=[
            pltpu.VMEM((EW,), _i32),
            pltpu.VMEM((EW,), _i32),
            pltpu.VMEM((NPAD,), _f32),
            pltpu.VMEM((NPAD,), _f32),
            pltpu.VMEM((NT,), _f32),
            pltpu.VMEM((NT,), _f32),
            pltpu.VMEM((NT,), _f32),
            pltpu.VMEM((NT,), _f32),
            pltpu.VMEM((NT,), _f32),
            pltpu.VMEM((NT,), _f32),
            pltpu.VMEM((32,), _f32),
            pltpu.VMEM((N_ITEMS,), _i32),
            pltpu.VMEM((N_ITEMS,), _f32),
            pltpu.VMEM_SHARED((NSUB, NPAD), _f32),
            pltpu.VMEM_SHARED((NPAD,), _f32),
        ],
    )
    probs = sc(z1d, src1d, dst1d, item1d, betas1d)
    return jax.nn.relu(domain_prior) * probs.reshape(N_ITEMS, 1)


# speedup vs baseline: 106.0589x; 1.4811x over previous
import jax
import jax.numpy as jnp
from jax import lax
from jax.experimental import pallas as pl
from jax.experimental.pallas import tpu as pltpu
from jax.experimental.pallas import tpu_sc as plsc

_f32 = jnp.float32
_i32 = jnp.int32

D = 128
N_USERS = 4000
N_ITEMS = 6000
N = N_USERS + N_ITEMS
E = 320000
NPAD = 10240
NSUB = 16
NT = NPAD // NSUB
ECH = 1256
EW = ECH * 16
EPAD = EW * NSUB
ICH = N_ITEMS // 16


def _dense_body(x_ref, w0_ref, w1_ref, wl_ref, b0_ref, b1_ref, bl_ref,
                z_ref, betas_ref):
    w2 = jnp.dot(w1_ref[...], wl_ref[...], preferred_element_type=_f32)
    w1v = jnp.dot(w0_ref[...], w2, preferred_element_type=_f32)
    z_ref[...] = jnp.dot(x_ref[...], w1v, preferred_element_type=_f32)
    beta1 = jnp.dot(b0_ref[...], w2, preferred_element_type=_f32)
    beta2 = jnp.dot(b1_ref[...], wl_ref[...], preferred_element_type=_f32)
    beta2 = beta2 + bl_ref[...]
    betas_ref[...] = jnp.concatenate(
        [jnp.broadcast_to(beta1, (1, D)), jnp.broadcast_to(beta2, (1, D))],
        axis=0)


def _fast_rsqrt(d):
    i = plsc.bitcast(d, _i32)
    y = plsc.bitcast(jnp.int32(0x5F3759DF) - (i >> 1), _f32)
    for _ in range(3):
        y = y * (1.5 - 0.5 * d * y * y)
    return y


def _sc_body(z_hbm, src_hbm, dst_hbm, item_hbm, betas_hbm,
             out_hbm,
             esrc, edst, tbl, acc, ptmp, rsl, zsl, dinvsl, thsl, vhsl, bb,
             items_v, res_v, sh_all, sh_tbl):
    t = lax.axis_index("s")
    core = lax.axis_index("c")
    ones = jnp.full((16,), 1.0, _f32)

    pltpu.sync_copy(src_hbm.at[pl.ds(t * EW, EW)], esrc)
    pltpu.sync_copy(dst_hbm.at[pl.ds(t * EW, EW)], edst)
    pltpu.sync_copy(z_hbm.at[pl.ds(t * NT, NT)], zsl)
    pltpu.sync_copy(betas_hbm, bb)

    def zero_acc():
        @plsc.parallel_loop(0, NPAD // 16, unroll=8)
        def _(r):
            acc[pl.ds(r * 16, 16)] = jnp.zeros((16,), _f32)

    def scatter_pass(gather_from_tbl):
        zero_acc()

        @plsc.parallel_loop(0, ECH, unroll=8)
        def _(i):
            di = edst[pl.ds(i * 16, 16)]
            if gather_from_tbl:
                si = esrc[pl.ds(i * 16, 16)]
                vals = plsc.load_gather(tbl, [si])
            else:
                vals = ones
            plsc.addupdate_scatter(acc, [di], vals)

    def reduce_acc():
        pltpu.sync_copy(acc, sh_all.at[t])
        plsc.subcore_barrier()
        pltpu.sync_copy(sh_all.at[0, pl.ds(t * NT, NT)], rsl)

        def rb(p, c):
            pltpu.sync_copy(sh_all.at[p, pl.ds(t * NT, NT)], ptmp)

            @plsc.parallel_loop(0, NT // 16, unroll=8)
            def _(r):
                o = r * 16
                rsl[pl.ds(o, 16)] = rsl[pl.ds(o, 16)] + ptmp[pl.ds(o, 16)]
            return c
        lax.fori_loop(1, NSUB, rb, 0)
        plsc.subcore_barrier()

    def publish_and_reload(slice_ref):
        pltpu.sync_copy(slice_ref, sh_tbl.at[pl.ds(t * NT, NT)])
        plsc.subcore_barrier()
        pltpu.sync_copy(sh_tbl, tbl)

    scatter_pass(gather_from_tbl=False)
    reduce_acc()

    @plsc.parallel_loop(0, NT // 16, unroll=4)
    def _(r):
        o = r * 16
        dv = _fast_rsqrt(rsl[pl.ds(o, 16)] + 1.0)
        dinvsl[pl.ds(o, 16)] = dv
        thsl[pl.ds(o, 16)] = zsl[pl.ds(o, 16)] * dv
    publish_and_reload(thsl)

    scatter_pass(gather_from_tbl=True)
    reduce_acc()
    b1v = bb[pl.ds(0, 16)]

    @plsc.parallel_loop(0, NT // 16, unroll=4)
    def _(r):
        o = r * 16
        vv = dinvsl[pl.ds(o, 16)] * (rsl[pl.ds(o, 16)] + thsl[pl.ds(o, 16)])
        vhsl[pl.ds(o, 16)] = (vv + b1v) * dinvsl[pl.ds(o, 16)]
    publish_and_reload(vhsl)

    scatter_pass(gather_from_tbl=True)
    reduce_acc()
    b2v = bb[pl.ds(16, 16)]

    @plsc.parallel_loop(0, NT // 16, unroll=4)
    def _(r):
        o = r * 16
        sv = dinvsl[pl.ds(o, 16)] * (rsl[pl.ds(o, 16)] + vhsl[pl.ds(o, 16)])
        thsl[pl.ds(o, 16)] = sv + b2v
    pltpu.sync_copy(thsl, sh_tbl.at[pl.ds(t * NT, NT)])
    plsc.subcore_barrier()

    @pl.when(jnp.logical_and(t == 0, core == 0))
    def _final():
        pltpu.sync_copy(sh_tbl, tbl)
        pltpu.sync_copy(item_hbm, items_v)

        @plsc.parallel_loop(0, ICH, unroll=8,
                            carry=jnp.full((16,), -1e30, _f32))
        def m(i, mc):
            it = items_v[pl.ds(i * 16, 16)]
            sv = plsc.load_gather(tbl, [it])
            res_v[pl.ds(i * 16, 16)] = sv
            return jnp.maximum(mc, sv)
        gm = jnp.max(m)

        @plsc.parallel_loop(0, ICH, unroll=8, carry=jnp.zeros((16,), _f32))
        def ssum(i, sc_):
            o = i * 16
            e = jnp.exp(res_v[pl.ds(o, 16)] - gm)
            res_v[pl.ds(o, 16)] = e
            return sc_ + e
        invv = jnp.ones((16,), _f32) / jnp.broadcast_to(jnp.sum(ssum), (16,))

        @plsc.parallel_loop(0, ICH, unroll=8)
        def _(i):
            o = i * 16
            res_v[pl.ds(o, 16)] = res_v[pl.ds(o, 16)] * invv
        pltpu.sync_copy(res_v, out_hbm)


def kernel(item, edge_index, user_embedding_weight, source_item_embedding_weight,
           W0, b0, W1, b1, Wl, bl, domain_prior):
    x = jnp.concatenate([user_embedding_weight, source_item_embedding_weight],
                        axis=0)
    z, betas = pl.pallas_call(
        _dense_body,
        out_shape=[jax.ShapeDtypeStruct((N, 1), _f32),
                   jax.ShapeDtypeStruct((2, D), _f32)],
    )(x, W0, W1, Wl, b0.reshape(1, D), b1.reshape(1, D), bl.reshape(1, 1))

    z1d = jnp.pad(z[:, 0], (0, NPAD - N))
    src1d = jnp.pad(edge_index[0].astype(_i32), (0, EPAD - E),
                    constant_values=N)
    dst1d = jnp.pad(edge_index[1].astype(_i32), (0, EPAD - E),
                    constant_values=N)
    item1d = item.astype(_i32)
    betas1d = betas[:, :16].reshape(32)

    sc = pl.kernel(
        _sc_body,
        out_type=jax.ShapeDtypeStruct((N_ITEMS,), _f32),
        mesh=plsc.VectorSubcoreMesh(core_axis_name="c", subcore_axis_name="s"),
        compiler_params=pltpu.CompilerParams(needs_layout_passes=False),
        scratch_types=[
            pltpu.VMEM((EW,), _i32),
            pltpu.VMEM((EW,), _i32),
            pltpu.VMEM((NPAD,), _f32),
            pltpu.VMEM((NPAD,), _f32),
            pltpu.VMEM((NT,), _f32),
            pltpu.VMEM((NT,), _f32),
            pltpu.VMEM((NT,), _f32),
            pltpu.VMEM((NT,), _f32),
            pltpu.VMEM((NT,), _f32),
            pltpu.VMEM((NT,), _f32),
            pltpu.VMEM((32,), _f32),
            pltpu.VMEM((N_ITEMS,), _i32),
            pltpu.VMEM((N_ITEMS,), _f32),
            pltpu.VMEM_SHARED((NSUB, NPAD), _f32),
            pltpu.VMEM_SHARED((NPAD,), _f32),
        ],
    )
    probs = sc(z1d, src1d, dst1d, item1d, betas1d)
    return jax.nn.relu(domain_prior) * probs.reshape(N_ITEMS, 1)
